# baseline (speedup 1.0000x reference)
import functools

import jax
import jax.numpy as jnp
from jax import lax
from jax.experimental import pallas as pl
from jax.experimental.pallas import tpu as pltpu
from jax.experimental.pallas import tpu_sc as plsc

N = 10000
E = 160000
LATENT = 128
FIRST = 32
B = 16

NSC = 2
NSUB = 16
NPAD = 10240
RPS = NPAD // NSUB
EB = 128
CK = 128
NBLK = E // EB
HBLK = NBLK // NSC
IPS = -(-HBLK // NSUB)


def _make_lap(C):
    nck = C // CK
    mesh = plsc.VectorSubcoreMesh(core_axis_name="c", subcore_axis_name="s")

    @functools.partial(
        pl.kernel,
        mesh=mesh,
        out_type=jax.ShapeDtypeStruct((NSC, NPAD, C), jnp.float32),
        scratch_types=[
            pltpu.VMEM_SHARED((NPAD, CK), jnp.float32),
            pltpu.VMEM((2, EB, CK), jnp.float32),
            pltpu.VMEM((2, EB), jnp.int32),
            pltpu.VMEM((2, EB), jnp.int32),
            pltpu.SemaphoreType.DMA,
            pltpu.SemaphoreType.DMA,
        ],
    )
    def lap(y_hbm, srcs_hbm, dsts_hbm, zeros_hbm, out_hbm,
            acc, gbuf, srcv, dstv, sem0, sem1):
        c = lax.axis_index("c")
        s = lax.axis_index("s")
        sems = (sem0, sem1)

        for p in range(nck):
            pltpu.sync_copy(zeros_hbm, acc.at[pl.ds(s * RPS, RPS)])
            plsc.subcore_barrier()

            def gcp(par):
                return pltpu.make_async_copy(
                    y_hbm.at[srcv.at[par], pl.ds(p * CK, CK)],
                    gbuf.at[par], sems[par])

            def start(i, par):
                kl = s + i * NSUB

                @pl.when(kl < HBLK)
                def _():
                    base = (c * HBLK + kl) * EB
                    pltpu.sync_copy(srcs_hbm.at[pl.ds(base, EB)],
                                    srcv.at[par])
                    pltpu.sync_copy(dsts_hbm.at[pl.ds(base, EB)],
                                    dstv.at[par])
                    gcp(par).start()

            def drain(i, par):
                @pl.when(s + i * NSUB < HBLK)
                def _():
                    gcp(par).wait()
                    pltpu.sync_copy(gbuf.at[par], acc.at[dstv.at[par]],
                                    add=True)

            start(0, 0)

            def step(j, carry):
                i0 = 2 * j
                start(i0 + 1, 1)
                drain(i0, 0)
                start(i0 + 2, 0)
                drain(i0 + 1, 1)
                return carry

            lax.fori_loop(0, IPS // 2, step, 0)

            plsc.subcore_barrier()
            pltpu.sync_copy(
                acc.at[pl.ds(s * RPS, RPS)],
                out_hbm.at[c, pl.ds(s * RPS, RPS), pl.ds(p * CK, CK)])

    return lap


def _proj_kernel(z_ref, w_ref, b_ref, o_ref):
    o_ref[...] = (jnp.dot(z_ref[...], w_ref[...],
                          preferred_element_type=jnp.float32)
                  + b_ref[...])


def _proj(z, w, b, bn):
    NF = w.shape[1]
    grid = (NF // bn,)
    return pl.pallas_call(
        _proj_kernel,
        grid=grid,
        in_specs=[pl.BlockSpec((B, LATENT), lambda j: (0, 0)),
                  pl.BlockSpec((LATENT, bn), lambda j: (0, j)),
                  pl.BlockSpec((1, bn), lambda j: (0, j))],
        out_specs=pl.BlockSpec((B, bn), lambda j: (0, j)),
        out_shape=jax.ShapeDtypeStruct((B, NF), jnp.float32),
    )(z, w, b)


def _scale_kernel(a_ref, s_ref, o_ref):
    o_ref[...] = a_ref[...] * s_ref[...]


def _scale(a, s, bm=1024):
    M, F = a.shape
    grid = (M // bm,)
    row = lambda i: (i, 0)
    return pl.pallas_call(
        _scale_kernel,
        grid=grid,
        in_specs=[pl.BlockSpec((bm, F), row), pl.BlockSpec((bm, 1), row)],
        out_specs=pl.BlockSpec((bm, F), row),
        out_shape=jax.ShapeDtypeStruct((M, F), jnp.float32),
    )(a, s)


def _scale2_kernel(a0_ref, a1_ref, s_ref, o_ref):
    o_ref[...] = (a0_ref[0] + a1_ref[0]) * s_ref[...]


def _scale2(a3, s, bm=1024):
    _, M, F = a3.shape
    grid = (M // bm,)
    return pl.pallas_call(
        _scale2_kernel,
        grid=grid,
        in_specs=[pl.BlockSpec((1, bm, F), lambda i: (0, i, 0)),
                  pl.BlockSpec((1, bm, F), lambda i: (1, i, 0)),
                  pl.BlockSpec((bm, 1), lambda i: (i, 0))],
        out_specs=pl.BlockSpec((bm, F), lambda i: (i, 0)),
        out_shape=jax.ShapeDtypeStruct((M, F), jnp.float32),
    )(a3, a3, s)


def _elu(x):
    return jnp.where(x > 0.0, x, jnp.exp(x) - 1.0)


def _combine_a_kernel(x_ref, sa0_ref, sa1_ref, sb0_ref, sb1_ref,
                      din_ref, dout_ref,
                      w0_ref, w1_ref, w2_ref, b_ref, o_ref, y_ref):
    x = x_ref[...]
    din = din_ref[...]
    t1 = -din * (sa0_ref[0] + sa1_ref[0])
    t2 = -2.0 * din * (sb0_ref[0] + sb1_ref[0]) - x
    acc = jnp.dot(x, w0_ref[...], preferred_element_type=jnp.float32)
    acc += jnp.dot(t1, w1_ref[...], preferred_element_type=jnp.float32)
    acc += jnp.dot(t2, w2_ref[...], preferred_element_type=jnp.float32)
    acc = _elu(acc + b_ref[...])
    o_ref[...] = acc
    y_ref[...] = acc * dout_ref[...]


def _combine_a(x, sa3, sb3, din, dout, W, b, bm):
    M, F = x.shape
    O = W.shape[2]
    grid = (M // bm,)
    full = lambda i: (0, 0)
    row = lambda i: (i, 0)
    h0 = lambda i: (0, i, 0)
    h1 = lambda i: (1, i, 0)
    return pl.pallas_call(
        _combine_a_kernel,
        grid=grid,
        in_specs=[pl.BlockSpec((bm, F), row),
                  pl.BlockSpec((1, bm, F), h0), pl.BlockSpec((1, bm, F), h1),
                  pl.BlockSpec((1, bm, F), h0), pl.BlockSpec((1, bm, F), h1),
                  pl.BlockSpec((bm, 1), row), pl.BlockSpec((bm, 1), row),
                  pl.BlockSpec((F, O), full), pl.BlockSpec((F, O), full),
                  pl.BlockSpec((F, O), full), pl.BlockSpec((1, O), full)],
        out_specs=[pl.BlockSpec((bm, O), row), pl.BlockSpec((bm, O), row)],
        out_shape=[jax.ShapeDtypeStruct((M, O), jnp.float32),
                   jax.ShapeDtypeStruct((M, O), jnp.float32)],
    )(x, sa3, sa3, sb3, sb3, din, dout, W[0], W[1], W[2], b)


def _combine_b_kernel(x_ref, sa0_ref, sa1_ref, sb0_ref, sb1_ref,
                      din_ref, dout_ref,
                      w0_ref, w1_ref, w2_ref, b_ref, wp1_ref, wp2_ref,
                      o_ref, yq_ref):
    x = x_ref[...]
    din = din_ref[...]
    t1 = -din * (sa0_ref[0] + sa1_ref[0])
    t2 = -2.0 * din * (sb0_ref[0] + sb1_ref[0]) - x
    acc = jnp.dot(x, w0_ref[...], preferred_element_type=jnp.float32)
    acc += jnp.dot(t1, w1_ref[...], preferred_element_type=jnp.float32)
    acc += jnp.dot(t2, w2_ref[...], preferred_element_type=jnp.float32)
    acc = _elu(acc + b_ref[...])
    o_ref[...] = acc
    p1 = jnp.dot(acc, wp1_ref[...], preferred_element_type=jnp.float32)
    p2 = jnp.dot(acc, wp2_ref[...], preferred_element_type=jnp.float32)
    yq_ref[...] = jnp.concatenate([p1, p2], axis=1) * dout_ref[...]


def _combine_b(x, sa3, sb3, din, dout, W, b, wp1, wp2, bm):
    M, F = x.shape
    O = W.shape[2]
    O2 = 2 * wp1.shape[1]
    grid = (M // bm,)
    full = lambda i: (0, 0)
    row = lambda i: (i, 0)
    h0 = lambda i: (0, i, 0)
    h1 = lambda i: (1, i, 0)
    return pl.pallas_call(
        _combine_b_kernel,
        grid=grid,
        in_specs=[pl.BlockSpec((bm, F), row),
                  pl.BlockSpec((1, bm, F), h0), pl.BlockSpec((1, bm, F), h1),
                  pl.BlockSpec((1, bm, F), h0), pl.BlockSpec((1, bm, F), h1),
                  pl.BlockSpec((bm, 1), row), pl.BlockSpec((bm, 1), row),
                  pl.BlockSpec((F, O), full), pl.BlockSpec((F, O), full),
                  pl.BlockSpec((F, O), full), pl.BlockSpec((1, O), full),
                  pl.BlockSpec((O, wp1.shape[1]), full),
                  pl.BlockSpec((O, wp2.shape[1]), full)],
        out_specs=[pl.BlockSpec((bm, O), row), pl.BlockSpec((bm, O2), row)],
        out_shape=[jax.ShapeDtypeStruct((M, O), jnp.float32),
                   jax.ShapeDtypeStruct((M, O2), jnp.float32)],
    )(x, sa3, sa3, sb3, sb3, din, dout, W[0], W[1], W[2], b, wp1, wp2)


def _final_kernel(x_ref, u1_ref, u3_ref, din_ref, wd_ref, b_ref, o_ref):
    din = din_ref[...]
    o_ref[...] = (jnp.dot(x_ref[...], wd_ref[...],
                          preferred_element_type=jnp.float32)
                  - din * u1_ref[...] - 2.0 * din * u3_ref[...] + b_ref[...])


def _final(x, u1, u3, din, wd, b, bm):
    M, F = x.shape
    O = wd.shape[1]
    grid = (M // bm,)
    full = lambda i: (0, 0)
    row = lambda i: (i, 0)
    return pl.pallas_call(
        _final_kernel,
        grid=grid,
        in_specs=[pl.BlockSpec((bm, F), row), pl.BlockSpec((bm, O), row),
                  pl.BlockSpec((bm, O), row), pl.BlockSpec((bm, 1), row),
                  pl.BlockSpec((F, O), full), pl.BlockSpec((1, O), full)],
        out_specs=pl.BlockSpec((bm, O), row),
        out_shape=jax.ShapeDtypeStruct((M, O), jnp.float32),
    )(x, u1, u3, din, wd, b)


def kernel(z, edge_index, proj_W, proj_b, W0, b0, W1, b1, W2, b2):
    src = edge_index[0].astype(jnp.int32)
    dst = edge_index[1].astype(jnp.int32)
    deg_out = jnp.zeros((N,), jnp.float32).at[src].add(1.0)
    deg_in = jnp.zeros((N,), jnp.float32).at[dst].add(1.0)
    dinv_out = lax.rsqrt(jnp.maximum(deg_out, 1.0))
    dinv_in = lax.rsqrt(jnp.maximum(deg_in, 1.0))
    ms = -(dinv_out * dinv_in)

    pad = (0, NPAD - N)
    din_col = jnp.repeat(jnp.pad(dinv_in, pad), B)[:, None]
    dout_col = jnp.repeat(jnp.pad(dinv_out, pad), B)[:, None]
    ms_col = jnp.repeat(jnp.pad(ms, pad), B)[:, None]
    zbuf = jnp.zeros((RPS, CK), jnp.float32)

    lap512 = _make_lap(512)
    lap1024 = _make_lap(1024)
    lap256 = _make_lap(256)
    M2 = NPAD * B

    x0 = _proj(z, proj_W, proj_b.reshape(1, -1), 2560)
    xt = jnp.swapaxes(x0.reshape(B, N, 32), 0, 1)
    x0r = jnp.pad(xt, (pad, (0, 0), (0, 0))).reshape(M2, 32)
    y0 = _scale(x0r, dout_col)
    sa3 = lap512(y0.reshape(NPAD, 512), src, dst, zbuf)
    sa3r = sa3.reshape(NSC, M2, 32)
    y1 = _scale2(sa3r, ms_col)
    sb3 = lap512(y1.reshape(NPAD, 512), src, dst, zbuf)
    sb3r = sb3.reshape(NSC, M2, 32)
    out0, ynext = _combine_a(x0r, sa3r, sb3r, din_col, dout_col,
                             W0, b0.reshape(1, -1), 1024)

    sa13 = lap1024(ynext.reshape(NPAD, 1024), src, dst, zbuf)
    sa13r = sa13.reshape(NSC, M2, 64)
    y11 = _scale2(sa13r, ms_col)
    sb13 = lap1024(y11.reshape(NPAD, 1024), src, dst, zbuf)
    sb13r = sb13.reshape(NSC, M2, 64)
    out1, yq = _combine_b(out0, sa13r, sb13r, din_col, dout_col,
                          W1, b1.reshape(1, -1), W2[1], W2[2], 1024)

    u123 = lap512(yq.reshape(NPAD, 512), src, dst, zbuf)
    u12n = (u123[0] + u123[1]).reshape(M2, 32)
    u1 = u12n[:, :16]
    u2 = u12n[:, 16:]
    yq3 = _scale(u2, ms_col)
    u3p3 = lap256(yq3.reshape(NPAD, 256), src, dst, zbuf)
    u3 = (u3p3[0] + u3p3[1]).reshape(M2, 16)
    out2 = _final(out1, u1, u3, din_col, W2[0] - W2[2],
                  b2.reshape(1, -1), 1024)
    return jnp.swapaxes(out2.reshape(NPAD, B, 16)[:N], 0, 1)
